# baseline (speedup 1.0000x reference)
import jax
import jax.numpy as jnp
from jax.experimental import pallas as pl
from jax.experimental.pallas import tpu as pltpu

_TT = 512


def _moe_body(x_ref, wg_ref, bg_ref, we_ref, be_ref,
              out_ref, scores_ref,
              xs_a, xs_b, wef_ref, bep_ref, s2_a, s2_b):
    D = x_ref.shape[1]
    E = wg_ref.shape[1]
    i = pl.program_id(0)

    def build(xs_t, s2_t):
        x = x_ref[...]
        logits = jnp.dot(x, wg_ref[...], preferred_element_type=jnp.float32)
        logits = logits + bg_ref[...]
        m = jnp.max(logits, axis=-1, keepdims=True)
        e = jnp.exp(logits - m)
        s = e / jnp.sum(e, axis=-1, keepdims=True)
        scores_ref[...] = s
        s2_t[...] = jnp.concatenate([s] * (128 // E),
                                    axis=1).astype(jnp.bfloat16)
        col = jax.lax.broadcasted_iota(jnp.int32, s.shape, 1)
        for kk in range(E):
            s_kk = jnp.sum(jnp.where(col == kk, s, 0.0), axis=1,
                           keepdims=True)
            xs_t[:, kk * D:(kk + 1) * D] = (x * s_kk).astype(jnp.bfloat16)

    def dot_tile(xs_t, s2_t):
        out_ref[...] = (
            jnp.dot(xs_t[...], wef_ref[...],
                    preferred_element_type=jnp.float32)
            + jnp.dot(s2_t[...], bep_ref[...],
                      preferred_element_type=jnp.float32)
        )

    @pl.when(i < E)
    def _cast_chunk():
        wef_ref[pl.ds(i * D, D), :] = we_ref[0].astype(jnp.bfloat16)

    @pl.when(i == 0)
    def _bias_pad():
        bep_ref[...] = jnp.concatenate(
            [be_ref[...].astype(jnp.bfloat16),
             jnp.zeros((128 - E, D), jnp.bfloat16)], axis=0)

    @pl.when(i == E - 1)
    def _prime():
        build(xs_a, s2_a)

    @pl.when((i >= E) & ((i - E) % 2 == 0))
    def _even():
        dot_tile(xs_a, s2_a)
        build(xs_b, s2_b)

    @pl.when((i >= E) & ((i - E) % 2 == 1))
    def _odd():
        dot_tile(xs_b, s2_b)
        build(xs_a, s2_a)


def kernel(x, Wg, bg, We, be):
    T, D = x.shape
    E = Wg.shape[1]
    n = T // _TT

    out, scores = pl.pallas_call(
        _moe_body,
        grid=(E + n,),
        in_specs=[
            pl.BlockSpec(
                (_TT, D),
                lambda i: (jnp.clip(i - E + 1, 0, n - 1), 0)),
            pl.BlockSpec((D, E), lambda i: (0, 0)),
            pl.BlockSpec((1, E), lambda i: (0, 0)),
            pl.BlockSpec((1, D, D),
                         lambda i: (jnp.minimum(i, E - 1), 0, 0)),
            pl.BlockSpec((E, D), lambda i: (0, 0)),
        ],
        out_specs=[
            pl.BlockSpec((_TT, D), lambda i: (jnp.maximum(i - E, 0), 0)),
            pl.BlockSpec(
                (_TT, E),
                lambda i: (jnp.clip(i - E + 1, 0, n - 1), 0)),
        ],
        out_shape=[
            jax.ShapeDtypeStruct((T, D), jnp.float32),
            jax.ShapeDtypeStruct((T, E), jnp.float32),
        ],
        scratch_shapes=[
            pltpu.VMEM((_TT, E * D), jnp.bfloat16),
            pltpu.VMEM((_TT, E * D), jnp.bfloat16),
            pltpu.VMEM((E * D, D), jnp.bfloat16),
            pltpu.VMEM((128, D), jnp.bfloat16),
            pltpu.VMEM((_TT, 128), jnp.bfloat16),
            pltpu.VMEM((_TT, 128), jnp.bfloat16),
        ],
        compiler_params=pltpu.CompilerParams(
            dimension_semantics=("arbitrary",),
        ),
    )(x, Wg, bg.reshape(1, E), We, be)
    return out, scores


# speedup vs baseline: 1.0688x; 1.0688x over previous
import jax
import jax.numpy as jnp
from jax.experimental import pallas as pl
from jax.experimental.pallas import tpu as pltpu

_TT = 512


def _moe_body(x_ref, wg_ref, bg_ref, we_ref, be_ref,
              out_ref, scores_ref, xs_ref, wef_ref, bep_ref, s2_ref):
    D = x_ref.shape[1]
    E = wg_ref.shape[1]
    TT = x_ref.shape[0]
    i = pl.program_id(0)

    @pl.when(i < E)
    def _cast_chunk():
        wef_ref[pl.ds(i * D, D), :] = we_ref[0].astype(jnp.bfloat16)

    @pl.when(i == 0)
    def _bias_pad():
        bep_ref[...] = jnp.concatenate(
            [be_ref[...].astype(jnp.bfloat16),
             jnp.zeros((128 - E, D), jnp.bfloat16)], axis=0)

    @pl.when(i >= E)
    def _tile():
        x = x_ref[...]
        logits = jnp.dot(x, wg_ref[...], preferred_element_type=jnp.float32)
        logits = logits + bg_ref[...]
        m = jnp.max(logits, axis=-1, keepdims=True)
        e = jnp.exp(logits - m)
        s = e / jnp.sum(e, axis=-1, keepdims=True)
        scores_ref[...] = s
        s2_ref[...] = jnp.concatenate([s] * (128 // E),
                                      axis=1).astype(jnp.bfloat16)
        col = jax.lax.broadcasted_iota(jnp.int32, (TT // 2, E), 1)
        for h in range(2):
            r = pl.ds(h * (TT // 2), TT // 2)
            sh = s[h * (TT // 2):(h + 1) * (TT // 2)]
            xh = x[h * (TT // 2):(h + 1) * (TT // 2)]
            for kk in range(E):
                s_kk = jnp.sum(jnp.where(col == kk, sh, 0.0), axis=1,
                               keepdims=True)
                xs_ref[r, kk * D:(kk + 1) * D] = (xh * s_kk).astype(
                    jnp.bfloat16)
            out_ref[r, :] = (
                jnp.dot(xs_ref[r, :], wef_ref[...],
                        preferred_element_type=jnp.float32)
                + jnp.dot(s2_ref[r, :], bep_ref[...],
                          preferred_element_type=jnp.float32)
            )


def kernel(x, Wg, bg, We, be):
    T, D = x.shape
    E = Wg.shape[1]
    n = T // _TT

    out, scores = pl.pallas_call(
        _moe_body,
        grid=(E + n,),
        in_specs=[
            pl.BlockSpec((_TT, D), lambda i: (jnp.maximum(i - E, 0), 0)),
            pl.BlockSpec((D, E), lambda i: (0, 0)),
            pl.BlockSpec((1, E), lambda i: (0, 0)),
            pl.BlockSpec((1, D, D),
                         lambda i: (jnp.minimum(i, E - 1), 0, 0)),
            pl.BlockSpec((E, D), lambda i: (0, 0)),
        ],
        out_specs=[
            pl.BlockSpec((_TT, D), lambda i: (jnp.maximum(i - E, 0), 0)),
            pl.BlockSpec((_TT, E), lambda i: (jnp.maximum(i - E, 0), 0)),
        ],
        out_shape=[
            jax.ShapeDtypeStruct((T, D), jnp.float32),
            jax.ShapeDtypeStruct((T, E), jnp.float32),
        ],
        scratch_shapes=[
            pltpu.VMEM((_TT, E * D), jnp.bfloat16),
            pltpu.VMEM((E * D, D), jnp.bfloat16),
            pltpu.VMEM((128, D), jnp.bfloat16),
            pltpu.VMEM((_TT, 128), jnp.bfloat16),
        ],
        compiler_params=pltpu.CompilerParams(
            dimension_semantics=("arbitrary",),
        ),
    )(x, Wg, bg.reshape(1, E), We, be)
    return out, scores


# speedup vs baseline: 1.0920x; 1.0217x over previous
import jax
import jax.numpy as jnp
from jax.experimental import pallas as pl
from jax.experimental.pallas import tpu as pltpu

_TT = 512


def _moe_body(x_ref, wg_ref, bg_ref, we_ref, be_ref,
              out_ref, scores_ref,
              xs_ref, wef_ref, bep_ref, s_scr, s2_scr):
    D = x_ref.shape[1]
    E = wg_ref.shape[1]
    TT = x_ref.shape[0]
    i = pl.program_id(0)

    @pl.when(i < E)
    def _prologue():
        wef_ref[pl.ds(i * D, D), :] = we_ref[0].astype(jnp.bfloat16)
        x = x_ref[...]
        logits = jnp.dot(x, wg_ref[...], preferred_element_type=jnp.float32)
        logits = logits + bg_ref[...]
        m = jnp.max(logits, axis=-1, keepdims=True)
        e = jnp.exp(logits - m)
        s = e / jnp.sum(e, axis=-1, keepdims=True)
        s_scr[pl.ds(i * TT, TT), :] = s
        s2_scr[pl.ds(i * TT, TT), :] = jnp.concatenate(
            [s] * (128 // E), axis=1).astype(jnp.bfloat16)

    @pl.when(i == 0)
    def _bias_pad():
        bep_ref[...] = jnp.concatenate(
            [be_ref[...].astype(jnp.bfloat16),
             jnp.zeros((128 - E, D), jnp.bfloat16)], axis=0)

    @pl.when(i >= E)
    def _tile():
        t = i - E
        x = x_ref[...]
        s = s_scr[pl.ds(t * TT, TT), :]
        scores_ref[...] = s
        s2 = s2_scr[pl.ds(t * TT, TT), :]
        col = jax.lax.broadcasted_iota(jnp.int32, (TT // 2, E), 1)
        for h in range(2):
            r = pl.ds(h * (TT // 2), TT // 2)
            sh = s[h * (TT // 2):(h + 1) * (TT // 2)]
            xh = x[h * (TT // 2):(h + 1) * (TT // 2)]
            for kk in range(E):
                s_kk = jnp.sum(jnp.where(col == kk, sh, 0.0), axis=1,
                               keepdims=True)
                xs_ref[r, kk * D:(kk + 1) * D] = (xh * s_kk).astype(
                    jnp.bfloat16)
            out_ref[r, :] = (
                jnp.dot(xs_ref[r, :], wef_ref[...],
                        preferred_element_type=jnp.float32)
                + jnp.dot(s2[h * (TT // 2):(h + 1) * (TT // 2)],
                          bep_ref[...],
                          preferred_element_type=jnp.float32)
            )


def kernel(x, Wg, bg, We, be):
    T, D = x.shape
    E = Wg.shape[1]
    n = T // _TT

    out, scores = pl.pallas_call(
        _moe_body,
        grid=(E + n,),
        in_specs=[
            pl.BlockSpec((_TT, D),
                         lambda i: (jnp.where(i < E, i, i - E), 0)),
            pl.BlockSpec((D, E), lambda i: (0, 0)),
            pl.BlockSpec((1, E), lambda i: (0, 0)),
            pl.BlockSpec((1, D, D),
                         lambda i: (jnp.minimum(i, E - 1), 0, 0)),
            pl.BlockSpec((E, D), lambda i: (0, 0)),
        ],
        out_specs=[
            pl.BlockSpec((_TT, D), lambda i: (jnp.maximum(i - E, 0), 0)),
            pl.BlockSpec((_TT, E), lambda i: (jnp.maximum(i - E, 0), 0)),
        ],
        out_shape=[
            jax.ShapeDtypeStruct((T, D), jnp.float32),
            jax.ShapeDtypeStruct((T, E), jnp.float32),
        ],
        scratch_shapes=[
            pltpu.VMEM((_TT, E * D), jnp.bfloat16),
            pltpu.VMEM((E * D, D), jnp.bfloat16),
            pltpu.VMEM((128, D), jnp.bfloat16),
            pltpu.VMEM((T, E), jnp.float32),
            pltpu.VMEM((T, 128), jnp.bfloat16),
        ],
        compiler_params=pltpu.CompilerParams(
            dimension_semantics=("arbitrary",),
        ),
    )(x, Wg, bg.reshape(1, E), We, be)
    return out, scores


# speedup vs baseline: 1.1023x; 1.0095x over previous
import jax
import jax.numpy as jnp
from jax.experimental import pallas as pl
from jax.experimental.pallas import tpu as pltpu

_TT = 512


def _moe_body(x_ref, wg_ref, bg_ref, we_ref, be_ref,
              out_ref, scores_ref,
              xs_ref, wef_ref, bep_ref, s_scr, s2_scr, x0b_ref):
    D = x_ref.shape[1]
    E = wg_ref.shape[1]
    TT = x_ref.shape[0]
    i = pl.program_id(0)

    @pl.when(i < E)
    def _prologue():
        wef_ref[pl.ds(i * D, D), :] = we_ref[0].astype(jnp.bfloat16)
        x = x_ref[...]
        logits = jnp.dot(x, wg_ref[...], preferred_element_type=jnp.float32)
        logits = logits + bg_ref[...]
        m = jnp.max(logits, axis=-1, keepdims=True)
        e = jnp.exp(logits - m)
        s = e / jnp.sum(e, axis=-1, keepdims=True)
        s_scr[pl.ds(i * TT, TT), :] = s
        s2_scr[pl.ds(i * TT, TT), :] = jnp.concatenate(
            [s] * (128 // E), axis=1).astype(jnp.bfloat16)

        @pl.when(i == 0)
        def _init0():
            x0b_ref[...] = x.astype(jnp.bfloat16)
            bep_ref[...] = jnp.concatenate(
                [be_ref[...].astype(jnp.bfloat16),
                 jnp.zeros((128 - E, D), jnp.bfloat16)], axis=0)
            scores_ref[...] = s

        s0 = s_scr[0:TT, :]
        col = jax.lax.broadcasted_iota(jnp.int32, (TT, E), 1)
        s_k0 = jnp.sum(jnp.where(col == i, s0, 0.0), axis=1, keepdims=True)
        y0 = jnp.dot(x0b_ref[...], wef_ref[pl.ds(i * D, D), :],
                     preferred_element_type=jnp.float32)

        @pl.when(i == 0)
        def _acc_first():
            out_ref[...] = y0 * s_k0

        @pl.when(i > 0)
        def _acc():
            out_ref[...] += y0 * s_k0

        @pl.when(i == E - 1)
        def _bias0():
            out_ref[...] += jnp.dot(s2_scr[0:TT, :], bep_ref[...],
                                    preferred_element_type=jnp.float32)

    @pl.when(i >= E)
    def _tile():
        t = i - E + 1
        x = x_ref[...]
        s = s_scr[pl.ds(t * TT, TT), :]
        scores_ref[...] = s
        s2 = s2_scr[pl.ds(t * TT, TT), :]
        col = jax.lax.broadcasted_iota(jnp.int32, (TT // 2, E), 1)
        for h in range(2):
            r = pl.ds(h * (TT // 2), TT // 2)
            sh = s[h * (TT // 2):(h + 1) * (TT // 2)]
            xh = x[h * (TT // 2):(h + 1) * (TT // 2)]
            for kk in range(E):
                s_kk = jnp.sum(jnp.where(col == kk, sh, 0.0), axis=1,
                               keepdims=True)
                xs_ref[r, kk * D:(kk + 1) * D] = (xh * s_kk).astype(
                    jnp.bfloat16)
            out_ref[r, :] = (
                jnp.dot(xs_ref[r, :], wef_ref[...],
                        preferred_element_type=jnp.float32)
                + jnp.dot(s2[h * (TT // 2):(h + 1) * (TT // 2)],
                          bep_ref[...],
                          preferred_element_type=jnp.float32)
            )


def kernel(x, Wg, bg, We, be):
    T, D = x.shape
    E = Wg.shape[1]
    n = T // _TT

    out, scores = pl.pallas_call(
        _moe_body,
        grid=(E + n - 1,),
        in_specs=[
            pl.BlockSpec((_TT, D),
                         lambda i: (jnp.where(i < E, i, i - E + 1), 0)),
            pl.BlockSpec((D, E), lambda i: (0, 0)),
            pl.BlockSpec((1, E), lambda i: (0, 0)),
            pl.BlockSpec((1, D, D),
                         lambda i: (jnp.minimum(i, E - 1), 0, 0)),
            pl.BlockSpec((E, D), lambda i: (0, 0)),
        ],
        out_specs=[
            pl.BlockSpec((_TT, D), lambda i: (jnp.maximum(i - E + 1, 0), 0)),
            pl.BlockSpec((_TT, E), lambda i: (jnp.maximum(i - E + 1, 0), 0)),
        ],
        out_shape=[
            jax.ShapeDtypeStruct((T, D), jnp.float32),
            jax.ShapeDtypeStruct((T, E), jnp.float32),
        ],
        scratch_shapes=[
            pltpu.VMEM((_TT, E * D), jnp.bfloat16),
            pltpu.VMEM((E * D, D), jnp.bfloat16),
            pltpu.VMEM((128, D), jnp.bfloat16),
            pltpu.VMEM((T, E), jnp.float32),
            pltpu.VMEM((T, 128), jnp.bfloat16),
            pltpu.VMEM((_TT, D), jnp.bfloat16),
        ],
        compiler_params=pltpu.CompilerParams(
            dimension_semantics=("arbitrary",),
        ),
    )(x, Wg, bg.reshape(1, E), We, be)
    return out, scores
